# baseline (speedup 1.0000x reference)
import jax
import jax.numpy as jnp
from jax.experimental import pallas as pl
from jax.experimental.pallas import tpu as pltpu


def _fused_kernel(hw, bt, x_ref, wb_ref, bb_ref, wf_ref, bf_ref, g_ref,
                  be_ref, rm_ref, rv_ref, wc_ref, bc_ref,
                  emb_ref, met_ref, clu_ref):
    cin = x_ref.shape[2]
    cemb = wb_ref.shape[1]
    x = x_ref[...]
    x2 = x.reshape(hw * bt, cin)
    wb_bf = wb_ref[...].astype(jnp.bfloat16)

    acc = jnp.dot(x2.astype(jnp.bfloat16), wb_bf,
                  preferred_element_type=jnp.float32) + bb_ref[...]
    emb_ref[...] = acc.reshape(hw, bt, cemb)

    x_mean = jnp.sum(x, axis=0) * (1.0 / hw)
    emb_mean = jnp.dot(x_mean, wb_ref[...],
                       preferred_element_type=jnp.float32) + bb_ref[...]
    feats = jnp.dot(emb_mean, wf_ref[...],
                    preferred_element_type=jnp.float32) + bf_ref[...]
    inv_f = jax.lax.rsqrt(
        jnp.maximum(jnp.sum(feats * feats, axis=-1, keepdims=True), 1e-24))
    metric = feats * inv_f

    s = g_ref[...] * jax.lax.rsqrt(rv_ref[...] + 1e-5)
    bn = metric * s + (be_ref[...] - rm_ref[...] * s)
    cluster = jnp.dot(bn, wc_ref[...],
                      preferred_element_type=jnp.float32) + bc_ref[...]
    inv_c = jax.lax.rsqrt(
        jnp.maximum(jnp.sum(cluster * cluster, axis=-1, keepdims=True), 1e-24))

    met_ref[...] = metric
    clu_ref[...] = cluster * inv_c


def kernel(x_nchw, w_base, b_base, w_feat, b_feat, bn_gamma, bn_beta,
           bn_rm, bn_rv, w_cl, b_cl):
    B, Cin, H, W = x_nchw.shape
    HW = H * W
    Cemb = w_base.shape[1]
    low_dim = w_feat.shape[1]
    n_cluster = w_cl.shape[1]

    bt = B
    for cand in (16, 8, 32, 40, 80):
        if B % cand == 0:
            bt = cand
            break
    n_tiles = B // bt

    x_hbc = jnp.transpose(x_nchw.reshape(B, Cin, HW), (2, 0, 1))

    flops = 2 * B * HW * Cin * Cemb + 2 * B * Cin * Cemb \
        + 2 * B * Cemb * low_dim + 2 * B * low_dim * n_cluster
    bytes_accessed = 4 * (B * HW * Cin + B * HW * Cemb + Cin * Cemb
                          + Cemb * low_dim + low_dim * n_cluster
                          + B * (low_dim + n_cluster))

    body = lambda *refs: _fused_kernel(HW, bt, *refs)
    emb_hbc, metric, cluster_n = pl.pallas_call(
        body,
        out_shape=(
            jax.ShapeDtypeStruct((HW, B, Cemb), jnp.float32),
            jax.ShapeDtypeStruct((B, low_dim), jnp.float32),
            jax.ShapeDtypeStruct((B, n_cluster), jnp.float32),
        ),
        grid=(n_tiles,),
        in_specs=[
            pl.BlockSpec((HW, bt, Cin), lambda i: (0, i, 0)),
            pl.BlockSpec((Cin, Cemb), lambda i: (0, 0)),
            pl.BlockSpec((1, Cemb), lambda i: (0, 0)),
            pl.BlockSpec((Cemb, low_dim), lambda i: (0, 0)),
            pl.BlockSpec((1, low_dim), lambda i: (0, 0)),
            pl.BlockSpec((1, low_dim), lambda i: (0, 0)),
            pl.BlockSpec((1, low_dim), lambda i: (0, 0)),
            pl.BlockSpec((1, low_dim), lambda i: (0, 0)),
            pl.BlockSpec((1, low_dim), lambda i: (0, 0)),
            pl.BlockSpec((low_dim, n_cluster), lambda i: (0, 0)),
            pl.BlockSpec((1, n_cluster), lambda i: (0, 0)),
        ],
        out_specs=(
            pl.BlockSpec((HW, bt, Cemb), lambda i: (0, i, 0)),
            pl.BlockSpec((bt, low_dim), lambda i: (i, 0)),
            pl.BlockSpec((bt, n_cluster), lambda i: (i, 0)),
        ),
        compiler_params=pltpu.CompilerParams(dimension_semantics=("parallel",)),
        cost_estimate=pl.CostEstimate(flops=flops, transcendentals=4 * B,
                                      bytes_accessed=bytes_accessed),
    )(x_hbc, w_base, b_base, w_feat, b_feat, bn_gamma, bn_beta,
      bn_rm, bn_rv, w_cl, b_cl)

    emb7x7 = jnp.transpose(emb_hbc, (1, 2, 0)).reshape(B, Cemb, H, W)
    return metric, cluster_n, emb7x7
